# Initial kernel scaffold; baseline (speedup 1.0000x reference)
import jax
import jax.numpy as jnp
from jax.experimental import pallas as pl


def kernel(x, emb_weight):
    raise NotImplementedError("write your pallas kernel here")


# speedup vs baseline: 1.1279x; 1.1279x over previous
import functools

import jax
import jax.numpy as jnp
from jax import lax
from jax.experimental import pallas as pl
from jax.experimental.pallas import tpu as pltpu
from jax.experimental.pallas import tpu_sc as plsc

_EMB_DIM = 64


@functools.lru_cache(maxsize=None)
def _build(B: int, D: int):
    info = plsc.get_sparse_core_info()
    NC, NS = info.num_cores, info.num_subcores
    NW = NC * NS
    assert B % NW == 0
    b_per_w = B // NW
    C = 512
    G = 128
    NCH = b_per_w // C
    GPC = C // G
    assert NCH * C == b_per_w and GPC * G == C

    mesh = plsc.VectorSubcoreMesh(core_axis_name="c", subcore_axis_name="s")

    @functools.partial(
        pl.kernel,
        out_type=jax.ShapeDtypeStruct((B, D), jnp.float32),
        mesh=mesh,
        scratch_types=[
            pltpu.VMEM((b_per_w,), jnp.int32),
            pltpu.VMEM((C, D), jnp.float32),
            pltpu.SemaphoreType.DMA,
        ],
        compiler_params=pltpu.CompilerParams(use_tc_tiling_on_sc=False),
    )
    def emb(table_hbm, idx_hbm, out_hbm, idx_v, rows_v, sem):
        wid = lax.axis_index("s") * NC + lax.axis_index("c")
        base = wid * b_per_w
        pltpu.sync_copy(idx_hbm.at[pl.ds(base, b_per_w)], idx_v)
        for i in range(NCH):
            handles = [
                pltpu.async_copy(
                    table_hbm.at[idx_v.at[pl.ds(i * C + g * G, G)]],
                    rows_v.at[pl.ds(g * G, G)],
                    sem,
                )
                for g in range(GPC)
            ]
            for h in handles:
                h.wait()
            pltpu.sync_copy(rows_v, out_hbm.at[pl.ds(base + i * C, C)])

    return emb


def kernel(x, emb_weight):
    B = x.shape[0] * x.shape[1]
    emb = _build(B, _EMB_DIM)
    flat_idx = x.reshape(-1).astype(jnp.int32)
    out = emb(emb_weight, flat_idx)
    return out.reshape(x.shape + (_EMB_DIM,))


# speedup vs baseline: 1.1294x; 1.0014x over previous
import functools

import jax
import jax.numpy as jnp
from jax import lax
from jax.experimental import pallas as pl
from jax.experimental.pallas import tpu as pltpu
from jax.experimental.pallas import tpu_sc as plsc

_EMB_DIM = 64


@functools.lru_cache(maxsize=None)
def _build(B: int, D: int):
    info = plsc.get_sparse_core_info()
    NC, NS = info.num_cores, info.num_subcores
    NW = NC * NS
    assert B % NW == 0
    b_per_w = B // NW
    C = 512
    G = 128
    NCH = b_per_w // C
    GPC = C // G
    assert NCH * C == b_per_w and GPC * G == C

    mesh = plsc.VectorSubcoreMesh(core_axis_name="c", subcore_axis_name="s")

    @functools.partial(
        pl.kernel,
        out_type=jax.ShapeDtypeStruct((B, D), jnp.float32),
        mesh=mesh,
        scratch_types=[
            pltpu.VMEM((b_per_w,), jnp.int32),
            pltpu.VMEM((C, D), jnp.float32),
            pltpu.VMEM((C, D), jnp.float32),
            pltpu.SemaphoreType.DMA,
            pltpu.SemaphoreType.DMA,
            pltpu.SemaphoreType.DMA,
            pltpu.SemaphoreType.DMA,
        ],
        compiler_params=pltpu.CompilerParams(use_tc_tiling_on_sc=False),
    )
    def emb(table_hbm, idx_hbm, out_hbm, idx_v, rows0, rows1, g0, g1, w0, w1):
        wid = lax.axis_index("s") * NC + lax.axis_index("c")
        base = wid * b_per_w
        bufs, gsems, wsems = [rows0, rows1], [g0, g1], [w0, w1]
        pltpu.sync_copy(idx_hbm.at[pl.ds(base, b_per_w)], idx_v)

        def issue_gathers(i):
            b = i % 2
            return [
                pltpu.async_copy(
                    table_hbm.at[idx_v.at[pl.ds(i * C + g * G, G)]],
                    bufs[b].at[pl.ds(g * G, G)],
                    gsems[b],
                )
                for g in range(GPC)
            ]

        ghandles = issue_gathers(0)
        whandles = [None] * NCH
        for i in range(NCH):
            b = i % 2
            for h in ghandles:
                h.wait()
            whandles[i] = pltpu.async_copy(
                bufs[b], out_hbm.at[pl.ds(base + i * C, C)], wsems[b]
            )
            if i + 1 < NCH:
                if i >= 1:
                    whandles[i - 1].wait()
                ghandles = issue_gathers(i + 1)
        whandles[NCH - 2].wait()
        whandles[NCH - 1].wait()

    return emb


def kernel(x, emb_weight):
    B = x.shape[0] * x.shape[1]
    emb = _build(B, _EMB_DIM)
    flat_idx = x.reshape(-1).astype(jnp.int32)
    out = emb(emb_weight, flat_idx)
    return out.reshape(x.shape + (_EMB_DIM,))
